# baseline (speedup 1.0000x reference)
import functools

import jax
import jax.numpy as jnp
from jax import lax
from jax.experimental import pallas as pl
from jax.experimental.pallas import tpu as pltpu
from jax.experimental.pallas import tpu_sc as plsc

N = 10000
E = 320000
IN_CH = 128
HID_CH = 32
OUT_CH = 16

NC, NS = 2, 16
NW = NC * NS
CHUNK = 128
NCHUNK = 80
EPW = NCHUNK * CHUNK
E_PAD = NW * EPW
NBUF = 2
N_PAD = 10240
DPT = N_PAD // NS

_mesh = plsc.VectorSubcoreMesh(core_axis_name="c", subcore_axis_name="s",
                               num_cores=NC, num_subcores=NS)
_sc_params = pltpu.CompilerParams(use_tc_tiling_on_sc=False)


@functools.partial(
    pl.kernel,
    out_type=jax.ShapeDtypeStruct((NC, N_PAD), jnp.float32),
    mesh=_mesh,
    compiler_params=_sc_params,
    scratch_types=[
        pltpu.VMEM((NCHUNK, CHUNK), jnp.int32),
        pltpu.VMEM((CHUNK,), jnp.float32),
        pltpu.VMEM((DPT,), jnp.float32),
        pltpu.VMEM_SHARED((N_PAD,), jnp.float32),
        pltpu.SemaphoreType.DMA,
    ],
)
def _deg_kernel(col_hbm, deg_hbm, col_v, ones_v, zb_v, deg_sh, sem0):
    c = lax.axis_index("c")
    s = lax.axis_index("s")
    wid = c * NS + s
    pltpu.sync_copy(col_hbm.at[wid], col_v)
    for i in range(CHUNK // 16):
        ones_v[pl.ds(16 * i, 16)] = jnp.ones((16,), jnp.float32)
    for i in range(DPT // 16):
        zb_v[pl.ds(16 * i, 16)] = jnp.zeros((16,), jnp.float32)
    pltpu.sync_copy(zb_v, deg_sh.at[pl.ds(s * DPT, DPT)])
    plsc.subcore_barrier()

    @pl.loop(0, NCHUNK)
    def _(j):
        pltpu.sync_copy(ones_v, deg_sh.at[col_v.at[j]], add=True)

    plsc.subcore_barrier()
    pltpu.sync_copy(deg_sh.at[pl.ds(s * DPT, DPT)],
                    deg_hbm.at[c, pl.ds(s * DPT, DPT)])


def _make_agg_kernel(d):
    @functools.partial(
        pl.kernel,
        out_type=jax.ShapeDtypeStruct((NC, N_PAD, d), jnp.float32),
        mesh=_mesh,
        compiler_params=_sc_params,
        scratch_types=[
            pltpu.VMEM((NCHUNK, CHUNK), jnp.int32),
            pltpu.VMEM((NCHUNK, CHUNK), jnp.int32),
            [pltpu.VMEM((CHUNK, d), jnp.float32) for _ in range(NBUF)],
            pltpu.VMEM_SHARED((N_PAD, d), jnp.float32),
            [pltpu.SemaphoreType.DMA for _ in range(NBUF)],
            [pltpu.SemaphoreType.DMA for _ in range(NBUF)],
        ],
    )
    def k(z_hbm, zeros_hbm, row_hbm, col_hbm, agg_hbm,
          row_v, col_v, msg, agg_sh, gsem, ssem):
        c = lax.axis_index("c")
        s = lax.axis_index("s")
        wid = c * NS + s
        cp_r = pltpu.async_copy(row_hbm.at[wid], row_v, gsem[0])
        cp_c = pltpu.async_copy(col_hbm.at[wid], col_v, gsem[1])
        pltpu.sync_copy(zeros_hbm.at[pl.ds(s * DPT, DPT)],
                        agg_sh.at[pl.ds(s * DPT, DPT)])
        cp_r.wait()
        cp_c.wait()
        plsc.subcore_barrier()

        @pl.loop(0, NCHUNK // 2)
        def _(i):
            j = 2 * i
            g0 = pltpu.async_copy(z_hbm.at[row_v.at[j]], msg[0], gsem[0])
            g1 = pltpu.async_copy(z_hbm.at[row_v.at[j + 1]], msg[1], gsem[1])
            g0.wait()
            pltpu.sync_copy(msg[0], agg_sh.at[col_v.at[j]], add=True)
            g1.wait()
            pltpu.sync_copy(msg[1], agg_sh.at[col_v.at[j + 1]], add=True)

        plsc.subcore_barrier()
        pltpu.sync_copy(agg_sh.at[pl.ds(s * DPT, DPT)],
                        agg_hbm.at[c, pl.ds(s * DPT, DPT)])

    return k


_agg_hid = _make_agg_kernel(HID_CH)
_agg_out = _make_agg_kernel(OUT_CH)


def _scale_in_kernel(x, W1, degt):
    def body(x_ref, w_ref, deg_ref, z_ref, dis_ref):
        deg = 1.0 + deg_ref[:, 0:1] + deg_ref[:, 1:2]
        dis = lax.rsqrt(deg)
        xw = jnp.dot(x_ref[...], w_ref[...], preferred_element_type=jnp.float32)
        z_ref[...] = dis * xw
        dis_ref[...] = dis

    return pl.pallas_call(
        body,
        out_shape=(jax.ShapeDtypeStruct((N, HID_CH), jnp.float32),
                   jax.ShapeDtypeStruct((N, 1), jnp.float32)),
    )(x, W1, degt)


def _mid_kernel(agg1, z1, dis, W2, b1):
    def body(p_ref, z_ref, dis_ref, w_ref, b_ref, z2_ref):
        agg = p_ref[0] + p_ref[1] + z_ref[...]
        h = jnp.maximum(dis_ref[...] * agg + b_ref[...], 0.0)
        xw2 = jnp.dot(h, w_ref[...], preferred_element_type=jnp.float32)
        z2_ref[...] = dis_ref[...] * xw2

    return pl.pallas_call(
        body,
        out_shape=jax.ShapeDtypeStruct((N, OUT_CH), jnp.float32),
    )(agg1, z1, dis, W2, b1)


def _final_kernel(agg2, z2, dis, b2):
    def body(q_ref, z2_ref, dis_ref, b_ref, out_ref):
        out_ref[...] = dis_ref[...] * (q_ref[0] + q_ref[1] + z2_ref[...]) + b_ref[...]

    return pl.pallas_call(
        body,
        out_shape=jax.ShapeDtypeStruct((N, OUT_CH), jnp.float32),
    )(agg2, z2, dis, b2)


def kernel(x, edge_index, W1, b1, W2, b2):
    row = edge_index[0].astype(jnp.int32)
    col = edge_index[1].astype(jnp.int32)
    rowp = jnp.concatenate(
        [row, jnp.zeros((E_PAD - E,), jnp.int32)]).reshape(NW, NCHUNK, CHUNK)
    colp = jnp.concatenate(
        [col, jnp.full((E_PAD - E,), N, jnp.int32)]).reshape(NW, NCHUNK, CHUNK)

    degp = _deg_kernel(colp)
    degt = degp[:, :N].T
    z1, dis = _scale_in_kernel(x, W1, degt)

    zeros_hid = jnp.zeros((N_PAD, HID_CH), jnp.float32)
    agg1 = _agg_hid(z1, zeros_hid, rowp, colp)[:, :N]
    z2 = _mid_kernel(agg1, z1, dis, W2, b1.reshape(1, HID_CH))

    zeros_out = jnp.zeros((N_PAD, OUT_CH), jnp.float32)
    agg2 = _agg_out(z2, zeros_out, rowp, colp)[:, :N]
    return _final_kernel(agg2, z2, dis, b2.reshape(1, OUT_CH))


# speedup vs baseline: 1.0110x; 1.0110x over previous
import functools

import jax
import jax.numpy as jnp
from jax import lax
from jax.experimental import pallas as pl
from jax.experimental.pallas import tpu as pltpu
from jax.experimental.pallas import tpu_sc as plsc

N = 10000
E = 320000
IN_CH = 128
HID_CH = 32
OUT_CH = 16

NC, NS = 2, 16
NW = NC * NS
CHUNK = 128
NCHUNK = 80
EPW = NCHUNK * CHUNK
E_PAD = NW * EPW
NBUF = 2
N_PAD = 10240
DPT = N_PAD // NS

_mesh = plsc.VectorSubcoreMesh(core_axis_name="c", subcore_axis_name="s",
                               num_cores=NC, num_subcores=NS)
_sc_params = pltpu.CompilerParams(use_tc_tiling_on_sc=False)


@functools.partial(
    pl.kernel,
    out_type=jax.ShapeDtypeStruct((NC, N_PAD), jnp.float32),
    mesh=_mesh,
    compiler_params=_sc_params,
    scratch_types=[
        pltpu.VMEM((NCHUNK, CHUNK), jnp.int32),
        pltpu.VMEM((CHUNK,), jnp.float32),
        pltpu.VMEM((DPT,), jnp.float32),
        pltpu.VMEM_SHARED((N_PAD,), jnp.float32),
        pltpu.SemaphoreType.DMA,
    ],
)
def _deg_kernel(col_hbm, deg_hbm, col_v, ones_v, zb_v, deg_sh, sem0):
    c = lax.axis_index("c")
    s = lax.axis_index("s")
    wid = c * NS + s
    pltpu.sync_copy(col_hbm.at[wid], col_v)
    for i in range(CHUNK // 16):
        ones_v[pl.ds(16 * i, 16)] = jnp.ones((16,), jnp.float32)
    for i in range(DPT // 16):
        zb_v[pl.ds(16 * i, 16)] = jnp.zeros((16,), jnp.float32)
    pltpu.sync_copy(zb_v, deg_sh.at[pl.ds(s * DPT, DPT)])
    plsc.subcore_barrier()

    @pl.loop(0, NCHUNK)
    def _(j):
        pltpu.sync_copy(ones_v, deg_sh.at[col_v.at[j]], add=True)

    plsc.subcore_barrier()
    pltpu.sync_copy(deg_sh.at[pl.ds(s * DPT, DPT)],
                    deg_hbm.at[c, pl.ds(s * DPT, DPT)])


def _make_agg_kernel(d):
    @functools.partial(
        pl.kernel,
        out_type=jax.ShapeDtypeStruct((NC, N_PAD, d), jnp.float32),
        mesh=_mesh,
        compiler_params=_sc_params,
        scratch_types=[
            pltpu.VMEM((NCHUNK, CHUNK), jnp.int32),
            pltpu.VMEM((NCHUNK, CHUNK), jnp.int32),
            [pltpu.VMEM((CHUNK, d), jnp.float32) for _ in range(NBUF)],
            pltpu.VMEM_SHARED((N_PAD, d), jnp.float32),
            [pltpu.SemaphoreType.DMA for _ in range(NBUF)],
            [pltpu.SemaphoreType.DMA for _ in range(NBUF)],
        ],
    )
    def k(z_hbm, zeros_hbm, row_hbm, col_hbm, agg_hbm,
          row_v, col_v, msg, agg_sh, gsem, ssem):
        c = lax.axis_index("c")
        s = lax.axis_index("s")
        wid = c * NS + s
        cp_r = pltpu.async_copy(row_hbm.at[wid], row_v, gsem[0])
        cp_c = pltpu.async_copy(col_hbm.at[wid], col_v, gsem[1])
        pltpu.sync_copy(zeros_hbm.at[pl.ds(s * DPT, DPT)],
                        agg_sh.at[pl.ds(s * DPT, DPT)])
        cp_r.wait()
        cp_c.wait()
        plsc.subcore_barrier()

        @pl.loop(0, NCHUNK // 2)
        def _(i):
            j = 2 * i
            g0 = pltpu.async_copy(z_hbm.at[row_v.at[j]], msg[0], gsem[0])
            g1 = pltpu.async_copy(z_hbm.at[row_v.at[j + 1]], msg[1], gsem[1])
            g0.wait()
            pltpu.sync_copy(msg[0], agg_sh.at[col_v.at[j]], add=True)
            g1.wait()
            pltpu.sync_copy(msg[1], agg_sh.at[col_v.at[j + 1]], add=True)

        plsc.subcore_barrier()
        pltpu.sync_copy(agg_sh.at[pl.ds(s * DPT, DPT)],
                        agg_hbm.at[c, pl.ds(s * DPT, DPT)])

    return k


_agg_hid = _make_agg_kernel(HID_CH)
_agg_out = _make_agg_kernel(OUT_CH)


def _scale_in_kernel(x, W1, degt):
    def body(x_ref, w_ref, deg_ref, z_ref, dis_ref):
        deg = 1.0 + deg_ref[:, 0:1] + deg_ref[:, 1:2]
        dis = lax.rsqrt(deg)
        xw = jnp.dot(x_ref[...], w_ref[...], preferred_element_type=jnp.float32)
        z_ref[...] = dis * xw
        dis_ref[...] = dis

    return pl.pallas_call(
        body,
        out_shape=(jax.ShapeDtypeStruct((N, HID_CH), jnp.float32),
                   jax.ShapeDtypeStruct((N, 1), jnp.float32)),
    )(x, W1, degt)


def _mid_kernel(agg1, z1, dis, W2, b1):
    def body(p_ref, z_ref, dis_ref, w_ref, b_ref, z2_ref):
        agg = p_ref[0] + p_ref[1] + z_ref[...]
        h = jnp.maximum(dis_ref[...] * agg + b_ref[...], 0.0)
        xw2 = jnp.dot(h, w_ref[...], preferred_element_type=jnp.float32)
        z2_ref[...] = dis_ref[...] * xw2

    return pl.pallas_call(
        body,
        out_shape=jax.ShapeDtypeStruct((N, OUT_CH), jnp.float32),
    )(agg1, z1, dis, W2, b1)


def _final_kernel(agg2, z2, dis, b2):
    def body(q_ref, z2_ref, dis_ref, b_ref, out_ref):
        out_ref[...] = dis_ref[...] * (q_ref[0] + q_ref[1] + z2_ref[...]) + b_ref[...]

    return pl.pallas_call(
        body,
        out_shape=jax.ShapeDtypeStruct((N, OUT_CH), jnp.float32),
    )(agg2, z2, dis, b2)


def kernel(x, edge_index, W1, b1, W2, b2):
    row = edge_index[0].astype(jnp.int32)
    col = edge_index[1].astype(jnp.int32)
    rowp = jnp.concatenate(
        [row, jnp.zeros((E_PAD - E,), jnp.int32)]).reshape(NW, NCHUNK, CHUNK)
    sink = N + jnp.arange(E_PAD - E, dtype=jnp.int32) % (N_PAD - N)
    colp = jnp.concatenate([col, sink]).reshape(NW, NCHUNK, CHUNK)

    degp = _deg_kernel(colp)
    degt = degp[:, :N].T
    z1, dis = _scale_in_kernel(x, W1, degt)

    zeros_hid = jnp.zeros((N_PAD, HID_CH), jnp.float32)
    agg1 = _agg_hid(z1, zeros_hid, rowp, colp)[:, :N]
    z2 = _mid_kernel(agg1, z1, dis, W2, b1.reshape(1, HID_CH))

    zeros_out = jnp.zeros((N_PAD, OUT_CH), jnp.float32)
    agg2 = _agg_out(z2, zeros_out, rowp, colp)[:, :N]
    return _final_kernel(agg2, z2, dis, b2.reshape(1, OUT_CH))


# speedup vs baseline: 1.4751x; 1.4591x over previous
import functools

import jax
import jax.numpy as jnp
from jax import lax
from jax.experimental import pallas as pl
from jax.experimental.pallas import tpu as pltpu
from jax.experimental.pallas import tpu_sc as plsc

N = 10000
E = 320000
IN_CH = 128
HID_CH = 32
OUT_CH = 16

NC, NS = 2, 16
NW = NC * NS
CHUNK = 128
NCHUNK = 80
EPW = NCHUNK * CHUNK
E_PAD = NW * EPW
NBUF = 2
N_PAD = 10240
DPT = N_PAD // NS

_mesh = plsc.VectorSubcoreMesh(core_axis_name="c", subcore_axis_name="s",
                               num_cores=NC, num_subcores=NS)
_sc_params = pltpu.CompilerParams(use_tc_tiling_on_sc=False)


@functools.partial(
    pl.kernel,
    out_type=jax.ShapeDtypeStruct((NC, N_PAD), jnp.float32),
    mesh=_mesh,
    compiler_params=_sc_params,
    scratch_types=[
        pltpu.VMEM((NCHUNK, CHUNK), jnp.int32),
        pltpu.VMEM((CHUNK,), jnp.float32),
        pltpu.VMEM((DPT,), jnp.float32),
        pltpu.VMEM_SHARED((N_PAD,), jnp.float32),
        pltpu.SemaphoreType.DMA,
    ],
)
def _deg_kernel(col_hbm, deg_hbm, col_v, ones_v, zb_v, deg_sh, sem0):
    c = lax.axis_index("c")
    s = lax.axis_index("s")
    wid = c * NS + s
    pltpu.sync_copy(col_hbm.at[wid], col_v)
    for i in range(CHUNK // 16):
        ones_v[pl.ds(16 * i, 16)] = jnp.ones((16,), jnp.float32)
    for i in range(DPT // 16):
        zb_v[pl.ds(16 * i, 16)] = jnp.zeros((16,), jnp.float32)
    pltpu.sync_copy(zb_v, deg_sh.at[pl.ds(s * DPT, DPT)])
    plsc.subcore_barrier()

    @pl.loop(0, NCHUNK)
    def _(j):
        pltpu.sync_copy(ones_v, deg_sh.at[col_v.at[j]], add=True)

    plsc.subcore_barrier()
    pltpu.sync_copy(deg_sh.at[pl.ds(s * DPT, DPT)],
                    deg_hbm.at[c, pl.ds(s * DPT, DPT)])


def _make_agg_kernel(d):
    @functools.partial(
        pl.kernel,
        out_type=jax.ShapeDtypeStruct((NC, N_PAD, d), jnp.float32),
        mesh=_mesh,
        compiler_params=_sc_params,
        scratch_types=[
            pltpu.VMEM((NCHUNK, CHUNK), jnp.int32),
            pltpu.VMEM((NCHUNK, CHUNK), jnp.int32),
            [pltpu.VMEM((CHUNK, d), jnp.float32) for _ in range(NBUF)],
            pltpu.VMEM_SHARED((N_PAD, d), jnp.float32),
            [pltpu.SemaphoreType.DMA for _ in range(NBUF)],
            [pltpu.SemaphoreType.DMA for _ in range(NBUF)],
        ],
    )
    def k(z_hbm, zeros_hbm, row_hbm, col_hbm, agg_hbm,
          row_v, col_v, msg, agg_sh, gsem, ssem):
        c = lax.axis_index("c")
        s = lax.axis_index("s")
        wid = c * NS + s
        cp_r = pltpu.async_copy(row_hbm.at[wid], row_v, gsem[0])
        cp_c = pltpu.async_copy(col_hbm.at[wid], col_v, gsem[1])
        pltpu.sync_copy(zeros_hbm.at[pl.ds(s * DPT, DPT)],
                        agg_sh.at[pl.ds(s * DPT, DPT)])
        cp_r.wait()
        cp_c.wait()
        plsc.subcore_barrier()

        @pl.loop(0, NCHUNK // 2)
        def _(i):
            j = 2 * i
            g0 = pltpu.async_copy(z_hbm.at[row_v.at[j]], msg[0], gsem[0])
            g1 = pltpu.async_copy(z_hbm.at[row_v.at[j + 1]], msg[1], gsem[1])
            g0.wait()
            pltpu.sync_copy(msg[0], agg_sh.at[col_v.at[j]], add=True)
            g1.wait()
            pltpu.sync_copy(msg[1], agg_sh.at[col_v.at[j + 1]], add=True)

        plsc.subcore_barrier()
        pltpu.sync_copy(agg_sh.at[pl.ds(s * DPT, DPT)],
                        agg_hbm.at[c, pl.ds(s * DPT, DPT)])

    return k


_agg_hid = _make_agg_kernel(HID_CH)
_agg_out = _make_agg_kernel(OUT_CH)


def _scale_in_kernel(x, W1, degt):
    def body(x_ref, w_ref, deg_ref, z_ref, dis_ref):
        deg = 1.0 + deg_ref[:, 0:1] + deg_ref[:, 1:2]
        dis = lax.rsqrt(deg)
        xw = jnp.dot(x_ref[...], w_ref[...], preferred_element_type=jnp.float32)
        z_ref[...] = dis * xw
        dis_ref[...] = dis

    return pl.pallas_call(
        body,
        out_shape=(jax.ShapeDtypeStruct((N, HID_CH), jnp.float32),
                   jax.ShapeDtypeStruct((N, 1), jnp.float32)),
    )(x, W1, degt)


def _mid_kernel(agg1, z1, dis, W2, b1):
    def body(p_ref, z_ref, dis_ref, w_ref, b_ref, z2_ref):
        agg = p_ref[0] + p_ref[1] + z_ref[...]
        h = jnp.maximum(dis_ref[...] * agg + b_ref[...], 0.0)
        xw2 = jnp.dot(h, w_ref[...], preferred_element_type=jnp.float32)
        z2_ref[...] = dis_ref[...] * xw2

    return pl.pallas_call(
        body,
        out_shape=jax.ShapeDtypeStruct((N, OUT_CH), jnp.float32),
    )(agg1, z1, dis, W2, b1)


def _final_kernel(agg2, z2, dis, b2):
    def body(q_ref, z2_ref, dis_ref, b_ref, out_ref):
        out_ref[...] = dis_ref[...] * (q_ref[0] + q_ref[1] + z2_ref[...]) + b_ref[...]

    return pl.pallas_call(
        body,
        out_shape=jax.ShapeDtypeStruct((N, OUT_CH), jnp.float32),
    )(agg2, z2, dis, b2)


def kernel(x, edge_index, W1, b1, W2, b2):
    row = edge_index[0].astype(jnp.int32)
    col = edge_index[1].astype(jnp.int32)
    pad_ar = jnp.arange(E_PAD - E, dtype=jnp.int32)
    rowp = jnp.concatenate([row, pad_ar % N]).reshape(NW, NCHUNK, CHUNK)
    sink = N + pad_ar % (N_PAD - N)
    colp = jnp.concatenate([col, sink]).reshape(NW, NCHUNK, CHUNK)

    degp = _deg_kernel(colp)
    degt = degp[:, :N].T
    z1, dis = _scale_in_kernel(x, W1, degt)

    zeros_hid = jnp.zeros((N_PAD, HID_CH), jnp.float32)
    agg1 = _agg_hid(z1, zeros_hid, rowp, colp)[:, :N]
    z2 = _mid_kernel(agg1, z1, dis, W2, b1.reshape(1, HID_CH))

    zeros_out = jnp.zeros((N_PAD, OUT_CH), jnp.float32)
    agg2 = _agg_out(z2, zeros_out, rowp, colp)[:, :N]
    return _final_kernel(agg2, z2, dis, b2.reshape(1, OUT_CH))


# speedup vs baseline: 1.5219x; 1.0317x over previous
import functools

import jax
import jax.numpy as jnp
from jax import lax
from jax.experimental import pallas as pl
from jax.experimental.pallas import tpu as pltpu
from jax.experimental.pallas import tpu_sc as plsc

N = 10000
E = 320000
IN_CH = 128
HID_CH = 32
OUT_CH = 16

NC, NS = 2, 16
NW = NC * NS
CHUNK = 128
NCHUNK = 80
EPW = NCHUNK * CHUNK
E_PAD = NW * EPW
NBUF = 2
N_PAD = 10240
DPT = N_PAD // NS

_mesh = plsc.VectorSubcoreMesh(core_axis_name="c", subcore_axis_name="s",
                               num_cores=NC, num_subcores=NS)
_sc_params = pltpu.CompilerParams(use_tc_tiling_on_sc=False)


@functools.partial(
    pl.kernel,
    out_type=jax.ShapeDtypeStruct((NC, N_PAD), jnp.float32),
    mesh=_mesh,
    compiler_params=_sc_params,
    scratch_types=[
        pltpu.VMEM((NCHUNK, CHUNK), jnp.int32),
        pltpu.VMEM((CHUNK,), jnp.float32),
        pltpu.VMEM((DPT,), jnp.float32),
        pltpu.VMEM_SHARED((N_PAD,), jnp.float32),
        pltpu.SemaphoreType.DMA,
    ],
)
def _deg_kernel(col_hbm, deg_hbm, col_v, ones_v, zb_v, deg_sh, sem0):
    c = lax.axis_index("c")
    s = lax.axis_index("s")
    wid = c * NS + s
    pltpu.sync_copy(col_hbm.at[wid], col_v)
    for i in range(CHUNK // 16):
        ones_v[pl.ds(16 * i, 16)] = jnp.ones((16,), jnp.float32)
    for i in range(DPT // 16):
        zb_v[pl.ds(16 * i, 16)] = jnp.zeros((16,), jnp.float32)
    pltpu.sync_copy(zb_v, deg_sh.at[pl.ds(s * DPT, DPT)])
    plsc.subcore_barrier()

    @pl.loop(0, NCHUNK)
    def _(j):
        pltpu.sync_copy(ones_v, deg_sh.at[col_v.at[j]], add=True)

    plsc.subcore_barrier()
    pltpu.sync_copy(deg_sh.at[pl.ds(s * DPT, DPT)],
                    deg_hbm.at[c, pl.ds(s * DPT, DPT)])


def _make_agg_kernel(d):
    @functools.partial(
        pl.kernel,
        out_type=jax.ShapeDtypeStruct((NC, N_PAD, d), jnp.float32),
        mesh=_mesh,
        compiler_params=_sc_params,
        scratch_types=[
            pltpu.VMEM((NCHUNK, CHUNK), jnp.int32),
            pltpu.VMEM((NCHUNK, CHUNK), jnp.int32),
            [pltpu.VMEM((CHUNK, d), jnp.float32) for _ in range(NBUF)],
            pltpu.VMEM_SHARED((N_PAD, d), jnp.float32),
            [pltpu.SemaphoreType.DMA for _ in range(NBUF)],
            [pltpu.SemaphoreType.DMA for _ in range(NBUF)],
        ],
    )
    def k(z_hbm, zeros_hbm, row_hbm, col_hbm, agg_hbm,
          row_v, col_v, msg, agg_sh, gsem, ssem):
        c = lax.axis_index("c")
        s = lax.axis_index("s")
        wid = c * NS + s
        cp_r = pltpu.async_copy(row_hbm.at[wid], row_v, gsem[0])
        cp_c = pltpu.async_copy(col_hbm.at[wid], col_v, gsem[1])
        pltpu.sync_copy(zeros_hbm.at[pl.ds(s * DPT, DPT)],
                        agg_sh.at[pl.ds(s * DPT, DPT)])
        cp_r.wait()
        cp_c.wait()
        plsc.subcore_barrier()

        @pl.loop(0, NCHUNK // 2)
        def _(i):
            j = 2 * i
            g0 = pltpu.async_copy(z_hbm.at[row_v.at[j]], msg[0], gsem[0])
            g1 = pltpu.async_copy(z_hbm.at[row_v.at[j + 1]], msg[1], gsem[1])
            g0.wait()
            pltpu.sync_copy(msg[0], agg_sh.at[col_v.at[j]], add=True)
            g1.wait()
            pltpu.sync_copy(msg[1], agg_sh.at[col_v.at[j + 1]], add=True)

        plsc.subcore_barrier()
        pltpu.sync_copy(agg_sh.at[pl.ds(s * DPT, DPT)],
                        agg_hbm.at[c, pl.ds(s * DPT, DPT)])

    return k


_agg_hid = _make_agg_kernel(HID_CH)
_agg_out = _make_agg_kernel(OUT_CH)


_ANY = pl.BlockSpec(memory_space=pl.ANY)
_VMEM = pl.BlockSpec(memory_space=pltpu.VMEM)


def _scale_in_kernel(x, W1, degt):
    def body(x_ref, w_ref, deg_ref, z_any, dis_ref, z_vmem):
        deg = 1.0 + deg_ref[:, 0:1] + deg_ref[:, 1:2]
        dis = lax.rsqrt(deg)
        xw = jnp.dot(x_ref[...], w_ref[...], preferred_element_type=jnp.float32)
        z_vmem[...] = dis * xw
        dis_ref[...] = dis
        pltpu.sync_copy(z_vmem, z_any)

    return pl.pallas_call(
        body,
        in_specs=[_VMEM, _VMEM, _VMEM],
        out_specs=(_ANY, _VMEM),
        out_shape=(jax.ShapeDtypeStruct((N, HID_CH), jnp.float32),
                   jax.ShapeDtypeStruct((N, 1), jnp.float32)),
        scratch_shapes=[pltpu.VMEM((N, HID_CH), jnp.float32)],
    )(x, W1, degt)


def _mid_kernel(agg1, z1, dis, W2, b1):
    def body(p_any, z_any, dis_ref, w_ref, b_ref, z2_any,
             p_vmem, z_vmem, z2_vmem):
        pltpu.sync_copy(p_any, p_vmem)
        pltpu.sync_copy(z_any, z_vmem)
        agg = p_vmem[0, :N] + p_vmem[1, :N] + z_vmem[...]
        h = jnp.maximum(dis_ref[...] * agg + b_ref[...], 0.0)
        xw2 = jnp.dot(h, w_ref[...], preferred_element_type=jnp.float32)
        z2_vmem[...] = dis_ref[...] * xw2
        pltpu.sync_copy(z2_vmem, z2_any)

    return pl.pallas_call(
        body,
        in_specs=[_ANY, _ANY, _VMEM, _VMEM, _VMEM],
        out_specs=_ANY,
        out_shape=jax.ShapeDtypeStruct((N, OUT_CH), jnp.float32),
        scratch_shapes=[pltpu.VMEM((NC, N_PAD, HID_CH), jnp.float32),
                        pltpu.VMEM((N, HID_CH), jnp.float32),
                        pltpu.VMEM((N, OUT_CH), jnp.float32)],
    )(agg1, z1, dis, W2, b1)


def _final_kernel(agg2, z2, dis, b2):
    def body(q_any, z2_any, dis_ref, b_ref, out_ref, q_vmem, z2_vmem):
        pltpu.sync_copy(q_any, q_vmem)
        pltpu.sync_copy(z2_any, z2_vmem)
        agg = q_vmem[0, :N] + q_vmem[1, :N] + z2_vmem[...]
        out_ref[...] = dis_ref[...] * agg + b_ref[...]

    return pl.pallas_call(
        body,
        in_specs=[_ANY, _ANY, _VMEM, _VMEM],
        out_specs=_VMEM,
        out_shape=jax.ShapeDtypeStruct((N, OUT_CH), jnp.float32),
        scratch_shapes=[pltpu.VMEM((NC, N_PAD, OUT_CH), jnp.float32),
                        pltpu.VMEM((N, OUT_CH), jnp.float32)],
    )(agg2, z2, dis, b2)


def kernel(x, edge_index, W1, b1, W2, b2):
    row = edge_index[0].astype(jnp.int32)
    col = edge_index[1].astype(jnp.int32)
    pad_ar = jnp.arange(E_PAD - E, dtype=jnp.int32)
    rowp = jnp.concatenate([row, pad_ar % N]).reshape(NW, NCHUNK, CHUNK)
    sink = N + pad_ar % (N_PAD - N)
    colp = jnp.concatenate([col, sink]).reshape(NW, NCHUNK, CHUNK)

    degp = _deg_kernel(colp)
    degt = degp[:, :N].T
    z1, dis = _scale_in_kernel(x, W1, degt)

    zeros_hid = jnp.zeros((N_PAD, HID_CH), jnp.float32)
    agg1 = _agg_hid(z1, zeros_hid, rowp, colp)
    z2 = _mid_kernel(agg1, z1, dis, W2, b1.reshape(1, HID_CH))

    zeros_out = jnp.zeros((N_PAD, OUT_CH), jnp.float32)
    agg2 = _agg_out(z2, zeros_out, rowp, colp)
    return _final_kernel(agg2, z2, dis, b2.reshape(1, OUT_CH))


# speedup vs baseline: 1.8891x; 1.2412x over previous
import functools

import jax
import jax.numpy as jnp
from jax import lax
from jax.experimental import pallas as pl
from jax.experimental.pallas import tpu as pltpu
from jax.experimental.pallas import tpu_sc as plsc

N = 10000
E = 320000
IN_CH = 128
HID_CH = 32
OUT_CH = 16

NC, NS = 2, 16
NW = NC * NS
CHUNK = 128
NCHUNK = 80
EPW = NCHUNK * CHUNK
E_PAD = NW * EPW
NBUF = 4
N_PAD = 10240
DPT = N_PAD // NS

_mesh = plsc.VectorSubcoreMesh(core_axis_name="c", subcore_axis_name="s",
                               num_cores=NC, num_subcores=NS)
_sc_params = pltpu.CompilerParams(use_tc_tiling_on_sc=False)


@functools.partial(
    pl.kernel,
    out_type=jax.ShapeDtypeStruct((NC, N_PAD), jnp.float32),
    mesh=_mesh,
    compiler_params=_sc_params,
    scratch_types=[
        pltpu.VMEM((NCHUNK, CHUNK), jnp.int32),
        pltpu.VMEM((CHUNK,), jnp.float32),
        pltpu.VMEM((DPT,), jnp.float32),
        pltpu.VMEM_SHARED((N_PAD,), jnp.float32),
        pltpu.SemaphoreType.DMA,
    ],
)
def _deg_kernel(col_hbm, deg_hbm, col_v, ones_v, zb_v, deg_sh, sem0):
    c = lax.axis_index("c")
    s = lax.axis_index("s")
    wid = c * NS + s
    pltpu.sync_copy(col_hbm.at[wid], col_v)
    for i in range(CHUNK // 16):
        ones_v[pl.ds(16 * i, 16)] = jnp.ones((16,), jnp.float32)
    for i in range(DPT // 16):
        zb_v[pl.ds(16 * i, 16)] = jnp.zeros((16,), jnp.float32)
    pltpu.sync_copy(zb_v, deg_sh.at[pl.ds(s * DPT, DPT)])
    plsc.subcore_barrier()

    @pl.loop(0, NCHUNK)
    def _(j):
        pltpu.sync_copy(ones_v, deg_sh.at[col_v.at[j]], add=True)

    plsc.subcore_barrier()
    pltpu.sync_copy(deg_sh.at[pl.ds(s * DPT, DPT)],
                    deg_hbm.at[c, pl.ds(s * DPT, DPT)])


def _make_agg_kernel(d):
    @functools.partial(
        pl.kernel,
        out_type=jax.ShapeDtypeStruct((NC, N_PAD, d), jnp.float32),
        mesh=_mesh,
        compiler_params=_sc_params,
        scratch_types=[
            pltpu.VMEM((NCHUNK, CHUNK), jnp.int32),
            pltpu.VMEM((NCHUNK, CHUNK), jnp.int32),
            [pltpu.VMEM((CHUNK, d), jnp.float32) for _ in range(NBUF)],
            pltpu.VMEM_SHARED((N_PAD, d), jnp.float32),
            [pltpu.SemaphoreType.DMA for _ in range(NBUF)],
            [pltpu.SemaphoreType.DMA for _ in range(NBUF)],
        ],
    )
    def k(z_hbm, zeros_hbm, row_hbm, col_hbm, agg_hbm,
          row_v, col_v, msg, agg_sh, gsem, ssem):
        c = lax.axis_index("c")
        s = lax.axis_index("s")
        wid = c * NS + s
        cp_r = pltpu.async_copy(row_hbm.at[wid], row_v, gsem[0])
        cp_c = pltpu.async_copy(col_hbm.at[wid], col_v, gsem[1])
        pltpu.sync_copy(zeros_hbm.at[pl.ds(s * DPT, DPT)],
                        agg_sh.at[pl.ds(s * DPT, DPT)])
        cp_r.wait()
        cp_c.wait()
        plsc.subcore_barrier()

        def gather(j, b):
            pltpu.async_copy(z_hbm.at[row_v.at[j]], msg[b], gsem[b])

        def gather_wait(j, b):
            pltpu.make_async_copy(z_hbm.at[row_v.at[j]], msg[b], gsem[b]).wait()

        def scatter(j, b):
            pltpu.async_copy(msg[b], agg_sh.at[col_v.at[j]], ssem[b], add=True)

        def scatter_wait(j, b):
            pltpu.make_async_copy(msg[b], agg_sh.at[col_v.at[j]], ssem[b]).wait()

        for b in range(NBUF):
            gather(b, b)

        @pl.loop(0, NCHUNK // NBUF - 1)
        def _(i):
            j0 = NBUF * i
            for b in range(NBUF):
                gather_wait(j0 + b, b)
                scatter(j0 + b, b)
            for b in range(NBUF):
                scatter_wait(j0 + b, b)
                gather(j0 + NBUF + b, b)

        for b in range(NBUF):
            j = NCHUNK - NBUF + b
            gather_wait(j, b)
            scatter(j, b)
        for b in range(NBUF):
            j = NCHUNK - NBUF + b
            scatter_wait(j, b)

        plsc.subcore_barrier()
        pltpu.sync_copy(agg_sh.at[pl.ds(s * DPT, DPT)],
                        agg_hbm.at[c, pl.ds(s * DPT, DPT)])

    return k


_agg_hid = _make_agg_kernel(HID_CH)
_agg_out = _make_agg_kernel(OUT_CH)


_ANY = pl.BlockSpec(memory_space=pl.ANY)
_VMEM = pl.BlockSpec(memory_space=pltpu.VMEM)


def _scale_in_kernel(x, W1, degt):
    def body(x_ref, w_ref, deg_ref, z_any, dis_ref, z_vmem):
        deg = 1.0 + deg_ref[:, 0:1] + deg_ref[:, 1:2]
        dis = lax.rsqrt(deg)
        xw = jnp.dot(x_ref[...], w_ref[...], preferred_element_type=jnp.float32)
        z_vmem[...] = dis * xw
        dis_ref[...] = dis
        pltpu.sync_copy(z_vmem, z_any)

    return pl.pallas_call(
        body,
        in_specs=[_VMEM, _VMEM, _VMEM],
        out_specs=(_ANY, _VMEM),
        out_shape=(jax.ShapeDtypeStruct((N, HID_CH), jnp.float32),
                   jax.ShapeDtypeStruct((N, 1), jnp.float32)),
        scratch_shapes=[pltpu.VMEM((N, HID_CH), jnp.float32)],
    )(x, W1, degt)


def _mid_kernel(agg1, z1, dis, W2, b1):
    def body(p_any, z_any, dis_ref, w_ref, b_ref, z2_any,
             p_vmem, z_vmem, z2_vmem):
        pltpu.sync_copy(p_any, p_vmem)
        pltpu.sync_copy(z_any, z_vmem)
        agg = p_vmem[0, :N] + p_vmem[1, :N] + z_vmem[...]
        h = jnp.maximum(dis_ref[...] * agg + b_ref[...], 0.0)
        xw2 = jnp.dot(h, w_ref[...], preferred_element_type=jnp.float32)
        z2_vmem[...] = dis_ref[...] * xw2
        pltpu.sync_copy(z2_vmem, z2_any)

    return pl.pallas_call(
        body,
        in_specs=[_ANY, _ANY, _VMEM, _VMEM, _VMEM],
        out_specs=_ANY,
        out_shape=jax.ShapeDtypeStruct((N, OUT_CH), jnp.float32),
        scratch_shapes=[pltpu.VMEM((NC, N_PAD, HID_CH), jnp.float32),
                        pltpu.VMEM((N, HID_CH), jnp.float32),
                        pltpu.VMEM((N, OUT_CH), jnp.float32)],
    )(agg1, z1, dis, W2, b1)


def _final_kernel(agg2, z2, dis, b2):
    def body(q_any, z2_any, dis_ref, b_ref, out_ref, q_vmem, z2_vmem):
        pltpu.sync_copy(q_any, q_vmem)
        pltpu.sync_copy(z2_any, z2_vmem)
        agg = q_vmem[0, :N] + q_vmem[1, :N] + z2_vmem[...]
        out_ref[...] = dis_ref[...] * agg + b_ref[...]

    return pl.pallas_call(
        body,
        in_specs=[_ANY, _ANY, _VMEM, _VMEM],
        out_specs=_VMEM,
        out_shape=jax.ShapeDtypeStruct((N, OUT_CH), jnp.float32),
        scratch_shapes=[pltpu.VMEM((NC, N_PAD, OUT_CH), jnp.float32),
                        pltpu.VMEM((N, OUT_CH), jnp.float32)],
    )(agg2, z2, dis, b2)


def kernel(x, edge_index, W1, b1, W2, b2):
    row = edge_index[0].astype(jnp.int32)
    col = edge_index[1].astype(jnp.int32)
    pad_ar = jnp.arange(E_PAD - E, dtype=jnp.int32)
    rowp = jnp.concatenate([row, pad_ar % N]).reshape(NW, NCHUNK, CHUNK)
    sink = N + pad_ar % (N_PAD - N)
    colp = jnp.concatenate([col, sink]).reshape(NW, NCHUNK, CHUNK)

    degp = _deg_kernel(colp)
    degt = degp[:, :N].T
    z1, dis = _scale_in_kernel(x, W1, degt)

    zeros_hid = jnp.zeros((N_PAD, HID_CH), jnp.float32)
    agg1 = _agg_hid(z1, zeros_hid, rowp, colp)
    z2 = _mid_kernel(agg1, z1, dis, W2, b1.reshape(1, HID_CH))

    zeros_out = jnp.zeros((N_PAD, OUT_CH), jnp.float32)
    agg2 = _agg_out(z2, zeros_out, rowp, colp)
    return _final_kernel(agg2, z2, dis, b2.reshape(1, OUT_CH))


# speedup vs baseline: 1.9993x; 1.0583x over previous
import functools

import jax
import jax.numpy as jnp
from jax import lax
from jax.experimental import pallas as pl
from jax.experimental.pallas import tpu as pltpu
from jax.experimental.pallas import tpu_sc as plsc

N = 10000
E = 320000
IN_CH = 128
HID_CH = 32
OUT_CH = 16

NC, NS = 2, 16
NW = NC * NS
CHUNK = 128
NCHUNK = 80
EPW = NCHUNK * CHUNK
E_PAD = NW * EPW
NBUF = 8
N_PAD = 10240
DPT = N_PAD // NS

_mesh = plsc.VectorSubcoreMesh(core_axis_name="c", subcore_axis_name="s",
                               num_cores=NC, num_subcores=NS)
_sc_params = pltpu.CompilerParams(use_tc_tiling_on_sc=False)


@functools.partial(
    pl.kernel,
    out_type=jax.ShapeDtypeStruct((NC, N_PAD), jnp.float32),
    mesh=_mesh,
    compiler_params=_sc_params,
    scratch_types=[
        pltpu.VMEM((NCHUNK, CHUNK), jnp.int32),
        pltpu.VMEM((CHUNK,), jnp.float32),
        pltpu.VMEM((DPT,), jnp.float32),
        pltpu.VMEM_SHARED((N_PAD,), jnp.float32),
        pltpu.SemaphoreType.DMA,
    ],
)
def _deg_kernel(col_hbm, deg_hbm, col_v, ones_v, zb_v, deg_sh, sem0):
    c = lax.axis_index("c")
    s = lax.axis_index("s")
    wid = c * NS + s
    pltpu.sync_copy(col_hbm.at[wid], col_v)
    for i in range(CHUNK // 16):
        ones_v[pl.ds(16 * i, 16)] = jnp.ones((16,), jnp.float32)
    for i in range(DPT // 16):
        zb_v[pl.ds(16 * i, 16)] = jnp.zeros((16,), jnp.float32)
    pltpu.sync_copy(zb_v, deg_sh.at[pl.ds(s * DPT, DPT)])
    plsc.subcore_barrier()

    @pl.loop(0, NCHUNK)
    def _(j):
        pltpu.sync_copy(ones_v, deg_sh.at[col_v.at[j]], add=True)

    plsc.subcore_barrier()
    pltpu.sync_copy(deg_sh.at[pl.ds(s * DPT, DPT)],
                    deg_hbm.at[c, pl.ds(s * DPT, DPT)])


def _make_agg_kernel(d):
    @functools.partial(
        pl.kernel,
        out_type=jax.ShapeDtypeStruct((NC, N_PAD, d), jnp.float32),
        mesh=_mesh,
        compiler_params=_sc_params,
        scratch_types=[
            pltpu.VMEM((NCHUNK, CHUNK), jnp.int32),
            pltpu.VMEM((NCHUNK, CHUNK), jnp.int32),
            [pltpu.VMEM((CHUNK, d), jnp.float32) for _ in range(NBUF)],
            pltpu.VMEM_SHARED((N_PAD, d), jnp.float32),
            [pltpu.SemaphoreType.DMA for _ in range(NBUF)],
            [pltpu.SemaphoreType.DMA for _ in range(NBUF)],
        ],
    )
    def k(z_hbm, zeros_hbm, row_hbm, col_hbm, agg_hbm,
          row_v, col_v, msg, agg_sh, gsem, ssem):
        c = lax.axis_index("c")
        s = lax.axis_index("s")
        wid = c * NS + s
        cp_r = pltpu.async_copy(row_hbm.at[wid], row_v, gsem[0])
        cp_c = pltpu.async_copy(col_hbm.at[wid], col_v, gsem[1])
        pltpu.sync_copy(zeros_hbm.at[pl.ds(s * DPT, DPT)],
                        agg_sh.at[pl.ds(s * DPT, DPT)])
        cp_r.wait()
        cp_c.wait()
        plsc.subcore_barrier()

        def gather(j, b):
            pltpu.async_copy(z_hbm.at[row_v.at[j]], msg[b], gsem[b])

        def gather_wait(j, b):
            pltpu.make_async_copy(z_hbm.at[row_v.at[j]], msg[b], gsem[b]).wait()

        def scatter(j, b):
            pltpu.async_copy(msg[b], agg_sh.at[col_v.at[j]], ssem[b], add=True)

        def scatter_wait(j, b):
            pltpu.make_async_copy(msg[b], agg_sh.at[col_v.at[j]], ssem[b]).wait()

        for b in range(NBUF):
            gather(b, b)

        @pl.loop(0, NCHUNK // NBUF - 1)
        def _(i):
            j0 = NBUF * i
            for b in range(NBUF):
                gather_wait(j0 + b, b)
                scatter(j0 + b, b)
            for b in range(NBUF):
                scatter_wait(j0 + b, b)
                gather(j0 + NBUF + b, b)

        for b in range(NBUF):
            j = NCHUNK - NBUF + b
            gather_wait(j, b)
            scatter(j, b)
        for b in range(NBUF):
            j = NCHUNK - NBUF + b
            scatter_wait(j, b)

        plsc.subcore_barrier()
        pltpu.sync_copy(agg_sh.at[pl.ds(s * DPT, DPT)],
                        agg_hbm.at[c, pl.ds(s * DPT, DPT)])

    return k


_agg_hid = _make_agg_kernel(HID_CH)
_agg_out = _make_agg_kernel(OUT_CH)


_ANY = pl.BlockSpec(memory_space=pl.ANY)
_VMEM = pl.BlockSpec(memory_space=pltpu.VMEM)


def _scale_in_kernel(x, W1, degt):
    def body(x_ref, w_ref, deg_ref, z_any, dis_ref, z_vmem):
        deg = 1.0 + deg_ref[:, 0:1] + deg_ref[:, 1:2]
        dis = lax.rsqrt(deg)
        xw = jnp.dot(x_ref[...], w_ref[...], preferred_element_type=jnp.float32)
        z_vmem[...] = dis * xw
        dis_ref[...] = dis
        pltpu.sync_copy(z_vmem, z_any)

    return pl.pallas_call(
        body,
        in_specs=[_VMEM, _VMEM, _VMEM],
        out_specs=(_ANY, _VMEM),
        out_shape=(jax.ShapeDtypeStruct((N, HID_CH), jnp.float32),
                   jax.ShapeDtypeStruct((N, 1), jnp.float32)),
        scratch_shapes=[pltpu.VMEM((N, HID_CH), jnp.float32)],
    )(x, W1, degt)


def _mid_kernel(agg1, z1, dis, W2, b1):
    def body(p_any, z_any, dis_ref, w_ref, b_ref, z2_any,
             p_vmem, z_vmem, z2_vmem):
        pltpu.sync_copy(p_any, p_vmem)
        pltpu.sync_copy(z_any, z_vmem)
        agg = p_vmem[0, :N] + p_vmem[1, :N] + z_vmem[...]
        h = jnp.maximum(dis_ref[...] * agg + b_ref[...], 0.0)
        xw2 = jnp.dot(h, w_ref[...], preferred_element_type=jnp.float32)
        z2_vmem[...] = dis_ref[...] * xw2
        pltpu.sync_copy(z2_vmem, z2_any)

    return pl.pallas_call(
        body,
        in_specs=[_ANY, _ANY, _VMEM, _VMEM, _VMEM],
        out_specs=_ANY,
        out_shape=jax.ShapeDtypeStruct((N, OUT_CH), jnp.float32),
        scratch_shapes=[pltpu.VMEM((NC, N_PAD, HID_CH), jnp.float32),
                        pltpu.VMEM((N, HID_CH), jnp.float32),
                        pltpu.VMEM((N, OUT_CH), jnp.float32)],
    )(agg1, z1, dis, W2, b1)


def _final_kernel(agg2, z2, dis, b2):
    def body(q_any, z2_any, dis_ref, b_ref, out_ref, q_vmem, z2_vmem):
        pltpu.sync_copy(q_any, q_vmem)
        pltpu.sync_copy(z2_any, z2_vmem)
        agg = q_vmem[0, :N] + q_vmem[1, :N] + z2_vmem[...]
        out_ref[...] = dis_ref[...] * agg + b_ref[...]

    return pl.pallas_call(
        body,
        in_specs=[_ANY, _ANY, _VMEM, _VMEM],
        out_specs=_VMEM,
        out_shape=jax.ShapeDtypeStruct((N, OUT_CH), jnp.float32),
        scratch_shapes=[pltpu.VMEM((NC, N_PAD, OUT_CH), jnp.float32),
                        pltpu.VMEM((N, OUT_CH), jnp.float32)],
    )(agg2, z2, dis, b2)


def kernel(x, edge_index, W1, b1, W2, b2):
    row = edge_index[0].astype(jnp.int32)
    col = edge_index[1].astype(jnp.int32)
    pad_ar = jnp.arange(E_PAD - E, dtype=jnp.int32)
    rowp = jnp.concatenate([row, pad_ar % N]).reshape(NW, NCHUNK, CHUNK)
    sink = N + pad_ar % (N_PAD - N)
    colp = jnp.concatenate([col, sink]).reshape(NW, NCHUNK, CHUNK)

    degp = _deg_kernel(colp)
    degt = degp[:, :N].T
    z1, dis = _scale_in_kernel(x, W1, degt)

    zeros_hid = jnp.zeros((N_PAD, HID_CH), jnp.float32)
    agg1 = _agg_hid(z1, zeros_hid, rowp, colp)
    z2 = _mid_kernel(agg1, z1, dis, W2, b1.reshape(1, HID_CH))

    zeros_out = jnp.zeros((N_PAD, OUT_CH), jnp.float32)
    agg2 = _agg_out(z2, zeros_out, rowp, colp)
    return _final_kernel(agg2, z2, dis, b2.reshape(1, OUT_CH))
